# baseline (speedup 1.0000x reference)
import functools

import jax
import jax.numpy as jnp
from jax import lax
from jax.experimental import pallas as pl
from jax.experimental.pallas import tpu as pltpu
from jax.experimental.pallas import tpu_sc as plsc

NC = 2
NS = 16
L = 16
NW = NC * NS


@functools.lru_cache(maxsize=None)
def _make_w2v(B, C, E, CB):
    assert B % (NW * CB) == 0 and E % L == 0
    b_per_w = B // NW
    steps = b_per_w // CB
    rows = CB * C
    ek = E // L

    mesh = plsc.VectorSubcoreMesh(core_axis_name="c", subcore_axis_name="s")

    @functools.partial(
        pl.kernel,
        out_type=jax.ShapeDtypeStruct((B * C,), jnp.float32),
        mesh=mesh,
        compiler_params=pltpu.CompilerParams(needs_layout_passes=False),
        scratch_types=[
            pltpu.VMEM((CB,), jnp.int32),
            pltpu.VMEM((CB,), jnp.int32),
            pltpu.VMEM((rows,), jnp.int32),
            pltpu.VMEM((rows,), jnp.int32),
            pltpu.VMEM((CB, E), jnp.float32),
            pltpu.VMEM((CB, E), jnp.float32),
            pltpu.VMEM((rows, E), jnp.float32),
            pltpu.VMEM((rows, E), jnp.float32),
            pltpu.VMEM((rows, L), jnp.float32),
            pltpu.VMEM((rows,), jnp.float32),
            pltpu.VMEM((rows,), jnp.float32),
            pltpu.SemaphoreType.DMA,
            pltpu.SemaphoreType.DMA,
            pltpu.SemaphoreType.DMA,
            pltpu.SemaphoreType.DMA,
            pltpu.SemaphoreType.DMA,
            pltpu.SemaphoreType.DMA,
        ],
    )
    def w2v(tgt_hbm, ctx_hbm, ttab_hbm, ctab_hbm, out_hbm,
            tidx0, tidx1, cidx0, cidx1, trows0, trows1, crows0, crows1,
            part_v, obuf0, obuf1, isem0, isem1, rsem0, rsem1, osem0, osem1):
        tidx = (tidx0, tidx1)
        cidx = (cidx0, cidx1)
        trows = (trows0, trows1)
        crows = (crows0, crows1)
        obuf = (obuf0, obuf1)
        isem = (isem0, isem1)
        rsem = (rsem0, rsem1)
        osem = (osem0, osem1)

        wid = lax.axis_index("s") * NC + lax.axis_index("c")
        base0 = wid * b_per_w

        def fire_idx(s):
            p = s % 2
            base = base0 + s * CB
            return (
                pltpu.async_copy(tgt_hbm.at[pl.ds(base, CB)],
                                 tidx[p], isem[p]),
                pltpu.async_copy(ctx_hbm.at[pl.ds(base * C, rows)],
                                 cidx[p], isem[p]),
            )

        def fire_rows(s, idx_descs):
            p = s % 2
            for d in idx_descs:
                d.wait()
            descs = [pltpu.async_copy(ttab_hbm.at[tidx[p]],
                                      trows[p], rsem[p])]
            for g in range(C):
                descs.append(pltpu.async_copy(
                    ctab_hbm.at[cidx[p].at[pl.ds(g * CB, CB)]],
                    crows[p].at[pl.ds(g * CB, CB)], rsem[p]))
            return descs

        def compute(s, row_descs, out_desc_old):
            p = s % 2
            for d in row_descs:
                d.wait()
            if out_desc_old is not None:
                out_desc_old.wait()
            tr = trows[p]
            cr = crows[p]

            @plsc.parallel_loop(0, CB, unroll=2)
            def dot_body(b):
                t = [tr[b, pl.ds(k * L, L)] for k in range(ek)]
                for c in range(C):
                    row = b * C + c
                    x = [t[k] * cr[row, pl.ds(k * L, L)] for k in range(ek)]
                    while len(x) > 1:
                        x = [x[i] + x[i + 1] for i in range(0, len(x) - 1, 2)] \
                            + ([x[-1]] if len(x) % 2 else [])
                    part_v[row, pl.ds(0, L)] = x[0]

            ob = obuf[p]

            @plsc.parallel_loop(0, rows // L, unroll=2)
            def red_body(g):
                rvec = g * L + lax.iota(jnp.int32, L)
                x = [plsc.load_gather(part_v,
                                      [rvec, jnp.full((L,), l, jnp.int32)])
                     for l in range(L)]
                while len(x) > 1:
                    x = [x[i] + x[i + 1] for i in range(0, len(x), 2)]
                ob[pl.ds(g * L, L)] = x[0]

            return pltpu.async_copy(
                ob, out_hbm.at[pl.ds((base0 + s * CB) * C, rows)],
                osem[p])

        idx_descs = [None] * steps
        row_descs = [None] * steps
        out_descs = [None] * steps
        idx_descs[0] = fire_idx(0)
        if steps > 1:
            idx_descs[1] = fire_idx(1)
        row_descs[0] = fire_rows(0, idx_descs[0])
        for s in range(steps):
            if s + 1 < steps:
                row_descs[s + 1] = fire_rows(s + 1, idx_descs[s + 1])
            old = out_descs[s - 2] if s >= 2 else None
            out_descs[s] = compute(s, row_descs[s], old)
            if s + 2 < steps:
                idx_descs[s + 2] = fire_idx(s + 2)
        for s in (steps - 2, steps - 1):
            if s >= 0 and out_descs[s] is not None:
                out_descs[s].wait()

    return w2v


def kernel(target, context, target_table, context_table):
    if target.ndim == 2:
        target = jnp.squeeze(target, axis=1)
    B, C = context.shape
    E = target_table.shape[1]
    w2v = _make_w2v(B, C, E, 32)
    out = w2v(target.astype(jnp.int32), context.reshape(-1),
              target_table, context_table)
    return out.reshape(B, C)


# speedup vs baseline: 1.3306x; 1.3306x over previous
import functools

import jax
import jax.numpy as jnp
from jax import lax
from jax.experimental import pallas as pl
from jax.experimental.pallas import tpu as pltpu
from jax.experimental.pallas import tpu_sc as plsc

NC = 2
NS = 16
L = 16
NW = NC * NS


@functools.lru_cache(maxsize=None)
def _make_w2v(B, C, E, CB):
    assert B % (NW * CB) == 0 and E % L == 0
    b_per_w = B // NW
    steps = b_per_w // CB
    rows = CB * C
    ek = E // L

    mesh = plsc.VectorSubcoreMesh(core_axis_name="c", subcore_axis_name="s")

    @functools.partial(
        pl.kernel,
        out_type=jax.ShapeDtypeStruct((B * C,), jnp.float32),
        mesh=mesh,
        compiler_params=pltpu.CompilerParams(needs_layout_passes=False),
        scratch_types=[
            pltpu.VMEM((CB,), jnp.int32),
            pltpu.VMEM((CB,), jnp.int32),
            pltpu.VMEM((rows,), jnp.int32),
            pltpu.VMEM((rows,), jnp.int32),
            pltpu.VMEM((CB, E), jnp.float32),
            pltpu.VMEM((CB, E), jnp.float32),
            pltpu.VMEM((rows, E), jnp.float32),
            pltpu.VMEM((rows, E), jnp.float32),
            pltpu.VMEM((rows, L), jnp.float32),
            pltpu.VMEM((rows,), jnp.float32),
            pltpu.VMEM((rows,), jnp.float32),
            pltpu.SemaphoreType.DMA,
            pltpu.SemaphoreType.DMA,
            pltpu.SemaphoreType.DMA,
            pltpu.SemaphoreType.DMA,
            pltpu.SemaphoreType.DMA,
            pltpu.SemaphoreType.DMA,
        ],
    )
    def w2v(tgt_hbm, ctx_hbm, ttab_hbm, ctab_hbm, out_hbm,
            tidx0, tidx1, cidx0, cidx1, trows0, trows1, crows0, crows1,
            part_v, obuf0, obuf1, isem0, isem1, rsem0, rsem1, osem0, osem1):
        tidx = (tidx0, tidx1)
        cidx = (cidx0, cidx1)
        trows = (trows0, trows1)
        crows = (crows0, crows1)
        obuf = (obuf0, obuf1)
        isem = (isem0, isem1)
        rsem = (rsem0, rsem1)
        osem = (osem0, osem1)

        wid = lax.axis_index("s") * NC + lax.axis_index("c")
        base0 = wid * b_per_w

        def fire_idx(s):
            p = s % 2
            base = base0 + s * CB
            descs = [pltpu.async_copy(tgt_hbm.at[pl.ds(base, CB)],
                                      tidx[p], isem[p])]
            for g in range(C):
                descs.append(pltpu.async_copy(
                    ctx_hbm.at[pl.ds(g * B + base, CB)],
                    cidx[p].at[pl.ds(g * CB, CB)], isem[p]))
            return descs

        def fire_rows(s, idx_descs):
            p = s % 2
            for d in idx_descs:
                d.wait()
            descs = [pltpu.async_copy(ttab_hbm.at[tidx[p]],
                                      trows[p], rsem[p])]
            for g in range(C):
                descs.append(pltpu.async_copy(
                    ctab_hbm.at[cidx[p].at[pl.ds(g * CB, CB)]],
                    crows[p].at[pl.ds(g * CB, CB)], rsem[p]))
            return descs

        def compute(s, row_descs, out_descs_old):
            p = s % 2
            for d in row_descs:
                d.wait()
            if out_descs_old is not None:
                for d in out_descs_old:
                    d.wait()
            tr = trows[p]
            cr = crows[p]

            def dot_body(b, carry):
                t = [tr[b, pl.ds(k * L, L)] for k in range(ek)]
                for c in range(C):
                    row = c * CB + b
                    x = [t[k] * cr[row, pl.ds(k * L, L)] for k in range(ek)]
                    while len(x) > 1:
                        x = [x[i] + x[i + 1] for i in range(0, len(x) - 1, 2)] \
                            + ([x[-1]] if len(x) % 2 else [])
                    part_v[row, pl.ds(0, L)] = x[0]
                return carry
            lax.fori_loop(0, CB, dot_body, 0)

            ob = obuf[p]

            def red_body(g, carry):
                rvec = g * L + lax.iota(jnp.int32, L)
                x = [plsc.load_gather(part_v,
                                      [rvec, jnp.full((L,), l, jnp.int32)])
                     for l in range(L)]
                while len(x) > 1:
                    x = [x[i] + x[i + 1] for i in range(0, len(x), 2)]
                ob[pl.ds(g * L, L)] = x[0]
                return carry
            lax.fori_loop(0, rows // L, red_body, 0)

            base = base0 + s * CB
            return [pltpu.async_copy(ob.at[pl.ds(g * CB, CB)],
                                     out_hbm.at[pl.ds(g * B + base, CB)],
                                     osem[p])
                    for g in range(C)]

        idx_descs = [None] * steps
        row_descs = [None] * steps
        out_descs = [None] * steps
        idx_descs[0] = fire_idx(0)
        if steps > 1:
            idx_descs[1] = fire_idx(1)
        row_descs[0] = fire_rows(0, idx_descs[0])
        for s in range(steps):
            if s + 1 < steps:
                row_descs[s + 1] = fire_rows(s + 1, idx_descs[s + 1])
            old = out_descs[s - 2] if s >= 2 else None
            out_descs[s] = compute(s, row_descs[s], old)
            if s + 2 < steps:
                idx_descs[s + 2] = fire_idx(s + 2)
        for s in (steps - 2, steps - 1):
            if s >= 0 and out_descs[s] is not None:
                for d in out_descs[s]:
                    d.wait()

    return w2v


def kernel(target, context, target_table, context_table):
    if target.ndim == 2:
        target = jnp.squeeze(target, axis=1)
    B, C = context.shape
    E = target_table.shape[1]
    w2v = _make_w2v(B, C, E, 32)
    out = w2v(target.astype(jnp.int32), context.T.reshape(-1),
              target_table, context_table)
    return out.reshape(C, B).T


# speedup vs baseline: 1.6096x; 1.2097x over previous
import functools

import jax
import jax.numpy as jnp
from jax import lax
from jax.experimental import pallas as pl
from jax.experimental.pallas import tpu as pltpu
from jax.experimental.pallas import tpu_sc as plsc

NC = 2
NS = 16
L = 16
NW = NC * NS


@functools.lru_cache(maxsize=None)
def _make_w2v(B, C, E, CB):
    assert B % (NW * CB) == 0 and E % L == 0
    b_per_w = B // NW
    steps = b_per_w // CB
    rows = CB * C
    ek = E // L

    mesh = plsc.VectorSubcoreMesh(core_axis_name="c", subcore_axis_name="s")

    @functools.partial(
        pl.kernel,
        out_type=jax.ShapeDtypeStruct((B * C,), jnp.float32),
        mesh=mesh,
        compiler_params=pltpu.CompilerParams(needs_layout_passes=False),
        scratch_types=[
            pltpu.VMEM((CB,), jnp.int32),
            pltpu.VMEM((CB,), jnp.int32),
            pltpu.VMEM((rows,), jnp.int32),
            pltpu.VMEM((rows,), jnp.int32),
            pltpu.VMEM((CB, E), jnp.float32),
            pltpu.VMEM((CB, E), jnp.float32),
            pltpu.VMEM((rows, E), jnp.float32),
            pltpu.VMEM((rows, E), jnp.float32),
            pltpu.VMEM((rows, L), jnp.float32),
            pltpu.VMEM((rows,), jnp.float32),
            pltpu.VMEM((rows,), jnp.float32),
            pltpu.SemaphoreType.DMA,
            pltpu.SemaphoreType.DMA,
            pltpu.SemaphoreType.DMA,
            pltpu.SemaphoreType.DMA,
            pltpu.SemaphoreType.DMA,
            pltpu.SemaphoreType.DMA,
        ],
    )
    def w2v(tgt_hbm, ctx_hbm, ttab_hbm, ctab_hbm, out_hbm,
            tidx0, tidx1, cidx0, cidx1, trows0, trows1, crows0, crows1,
            part_v, obuf0, obuf1, isem0, isem1, rsem0, rsem1, osem0, osem1):
        tidx = (tidx0, tidx1)
        cidx = (cidx0, cidx1)
        trows = (trows0, trows1)
        crows = (crows0, crows1)
        obuf = (obuf0, obuf1)
        isem = (isem0, isem1)
        rsem = (rsem0, rsem1)
        osem = (osem0, osem1)

        wid = lax.axis_index("s") * NC + lax.axis_index("c")
        base0 = wid * b_per_w

        def fire_idx(s, p):
            base = pl.multiple_of(base0 + s * CB, CB)
            pltpu.async_copy(tgt_hbm.at[pl.ds(base, CB)], tidx[p], isem[p])
            for g in range(C):
                pltpu.async_copy(ctx_hbm.at[pl.ds(g * B + base, CB)],
                                 cidx[p].at[pl.ds(g * CB, CB)], isem[p])

        def drain_idx(p):
            pltpu.make_async_copy(tgt_hbm.at[pl.ds(0, CB)],
                                  tidx[p], isem[p]).wait()
            for g in range(C):
                pltpu.make_async_copy(ctx_hbm.at[pl.ds(0, CB)],
                                      cidx[p].at[pl.ds(g * CB, CB)],
                                      isem[p]).wait()

        def fire_rows(p):
            pltpu.async_copy(ttab_hbm.at[tidx[p]], trows[p], rsem[p])
            for g in range(C):
                pltpu.async_copy(ctab_hbm.at[cidx[p].at[pl.ds(g * CB, CB)]],
                                 crows[p].at[pl.ds(g * CB, CB)], rsem[p])

        def drain_rows(p):
            pltpu.make_async_copy(ttab_hbm.at[tidx[p]],
                                  trows[p], rsem[p]).wait()
            for g in range(C):
                pltpu.make_async_copy(ctab_hbm.at[cidx[p].at[pl.ds(g * CB, CB)]],
                                      crows[p].at[pl.ds(g * CB, CB)],
                                      rsem[p]).wait()

        def compute_core(p):
            tr = trows[p]
            cr = crows[p]

            def dot_body(b, carry):
                t = [tr[b, pl.ds(k * L, L)] for k in range(ek)]
                for c in range(C):
                    row = c * CB + b
                    x = [t[k] * cr[row, pl.ds(k * L, L)] for k in range(ek)]
                    while len(x) > 1:
                        x = [x[i] + x[i + 1] for i in range(0, len(x) - 1, 2)] \
                            + ([x[-1]] if len(x) % 2 else [])
                    part_v[row, pl.ds(0, L)] = x[0]
                return carry
            lax.fori_loop(0, CB, dot_body, 0)

            ob = obuf[p]

            def red_body(g, carry):
                rvec = g * L + lax.iota(jnp.int32, L)
                x = [plsc.load_gather(part_v,
                                      [rvec, jnp.full((L,), l, jnp.int32)])
                     for l in range(L)]
                while len(x) > 1:
                    x = [x[i] + x[i + 1] for i in range(0, len(x), 2)]
                ob[pl.ds(g * L, L)] = x[0]
                return carry
            lax.fori_loop(0, rows // L, red_body, 0)

        def fire_out(s, p):
            base = pl.multiple_of(base0 + s * CB, CB)
            for g in range(C):
                pltpu.async_copy(obuf[p].at[pl.ds(g * CB, CB)],
                                 out_hbm.at[pl.ds(g * B + base, CB)],
                                 osem[p])

        def drain_out(p):
            for g in range(C):
                pltpu.make_async_copy(obuf[p].at[pl.ds(g * CB, CB)],
                                      out_hbm.at[pl.ds(0, CB)],
                                      osem[p]).wait()

        assert steps >= 6 and steps % 2 == 0
        fire_idx(0, 0)
        fire_idx(1, 1)
        drain_idx(0)
        fire_rows(0)

        drain_idx(1)
        fire_rows(1)
        drain_rows(0)
        fire_idx(2, 0)
        compute_core(0)
        fire_out(0, 0)
        drain_idx(0)
        fire_rows(0)
        drain_rows(1)
        fire_idx(3, 1)
        compute_core(1)
        fire_out(1, 1)

        def pipe_body(i, carry):
            s0 = 2 * i
            drain_idx(1)
            fire_rows(1)
            drain_rows(0)
            fire_idx(s0 + 2, 0)
            drain_out(0)
            compute_core(0)
            fire_out(s0, 0)
            drain_idx(0)
            fire_rows(0)
            drain_rows(1)
            fire_idx(s0 + 3, 1)
            drain_out(1)
            compute_core(1)
            fire_out(s0 + 1, 1)
            return carry
        lax.fori_loop(1, steps // 2 - 1, pipe_body, 0)

        drain_idx(1)
        fire_rows(1)
        drain_rows(0)
        drain_out(0)
        compute_core(0)
        fire_out(steps - 2, 0)
        drain_rows(1)
        drain_out(1)
        compute_core(1)
        fire_out(steps - 1, 1)
        drain_out(0)
        drain_out(1)

    return w2v


def kernel(target, context, target_table, context_table):
    if target.ndim == 2:
        target = jnp.squeeze(target, axis=1)
    B, C = context.shape
    E = target_table.shape[1]
    w2v = _make_w2v(B, C, E, 32)
    out = w2v(target.astype(jnp.int32), context.T.reshape(-1),
              target_table, context_table)
    return out.reshape(C, B).T
